# Initial kernel scaffold; baseline (speedup 1.0000x reference)
import jax
import jax.numpy as jnp
from jax.experimental import pallas as pl


def kernel(values, prefix_sum):
    raise NotImplementedError("write your pallas kernel here")


# speedup vs baseline: 5.6227x; 5.6227x over previous
import jax
import jax.numpy as jnp
from jax import lax
from jax.experimental import pallas as pl
from jax.experimental.pallas import tpu as pltpu
from jax.experimental.pallas import tpu_sc as plsc

N_TOKENS = 32768
N_SEGS = 16
NUM_SUBCORES = 16
CHUNK = N_TOKENS // NUM_SUBCORES
LANES = 16
VREGS = CHUNK // LANES
I32_MAX = jnp.iinfo(jnp.int32).max


def _sc_body(values_hbm, ps_hbm, pv_hbm, pp_hbm, out_hbm,
             vals_v, ps_v, res_v, res_p, buf_v, buf_p, out_v):
    sid = lax.axis_index("s")
    cid = lax.axis_index("c")
    base = sid * CHUNK

    pltpu.sync_copy(values_hbm.at[pl.ds(base, CHUNK)], vals_v)
    pltpu.sync_copy(ps_hbm, ps_v)

    iota = lax.iota(jnp.int32, LANES)
    neg_inf = jnp.float32(-jnp.inf)
    pos_inf = jnp.float32(jnp.inf)
    neg16 = jnp.full((LANES,), neg_inf, jnp.float32)
    inf16 = jnp.full((LANES,), pos_inf, jnp.float32)

    ps16 = ps_v[...]
    seg_max = neg16
    seg_pos = inf16
    for s in range(N_SEGS):
        end_s = ps16[s]
        start_s = ps16[s - 1] if s > 0 else jnp.int32(0)
        lo = jnp.maximum(start_s, base)
        hi = jnp.minimum(end_s, base + CHUNK)
        lo_c = jnp.clip(lo - base, 0, CHUNK)
        hi_c = jnp.clip(hi - base, 0, CHUNK)
        j0 = lo_c // LANES
        j1 = (hi_c + (LANES - 1)) // LANES

        def body(j, carry, lo=lo, hi=hi, start_s=start_s):
            mv, mp = carry
            v = vals_v[pl.ds(j * LANES, LANES)]
            pos = base + j * LANES + iota
            inside = (pos >= lo) & (pos < hi)
            v = jnp.where(inside, v, neg_inf)
            upd = v > mv
            mp = jnp.where(upd, (pos - start_s).astype(jnp.float32), mp)
            mv = jnp.where(upd, v, mv)
            return mv, mp

        mv, mp = lax.fori_loop(j0, j1, body, (neg16, inf16))
        m = mv
        for k in (8, 4, 2, 1):
            m = jnp.maximum(m, m.at[iota ^ k].get(mode="promise_in_bounds"))
        p = jnp.where(mv == m, mp, inf16)
        for k in (8, 4, 2, 1):
            p = jnp.minimum(p, p.at[iota ^ k].get(mode="promise_in_bounds"))
        seg_max = jnp.where(iota == s, m, seg_max)
        seg_pos = jnp.where(iota == s, p, seg_pos)

    res_v[...] = seg_max
    res_p[...] = seg_pos
    pltpu.sync_copy(res_v, pv_hbm.at[sid])
    pltpu.sync_copy(res_p, pp_hbm.at[sid])
    plsc.subcore_barrier()

    @pl.when((sid == 0) & (cid == 0))
    def _():
        pltpu.sync_copy(pv_hbm, buf_v)
        pltpu.sync_copy(pp_hbm, buf_p)
        gmax = neg16
        for i in range(NUM_SUBCORES):
            gmax = jnp.maximum(gmax, buf_v[i, :])
        gpos = inf16
        for i in range(NUM_SUBCORES):
            rv = buf_v[i, :]
            rp = buf_p[i, :]
            gpos = jnp.minimum(gpos, jnp.where(rv == gmax, rp, inf16))
        empty = gmax == neg16
        out_v[...] = jnp.where(empty, jnp.full((LANES,), I32_MAX, jnp.int32),
                               gpos.astype(jnp.int32))
        pltpu.sync_copy(out_v, out_hbm)


@jax.jit
def _jagged_argmax_sc(values, prefix_sum):
    mesh = plsc.VectorSubcoreMesh(
        core_axis_name="c", subcore_axis_name="s",
        num_cores=2, num_subcores=NUM_SUBCORES)
    _, _, out = pl.kernel(
        _sc_body,
        out_type=[
            jax.ShapeDtypeStruct((NUM_SUBCORES, LANES), jnp.float32),
            jax.ShapeDtypeStruct((NUM_SUBCORES, LANES), jnp.float32),
            jax.ShapeDtypeStruct((N_SEGS,), jnp.int32),
        ],
        mesh=mesh,
        scratch_types=[
            pltpu.VMEM((CHUNK,), jnp.float32),
            pltpu.VMEM((N_SEGS,), jnp.int32),
            pltpu.VMEM((LANES,), jnp.float32),
            pltpu.VMEM((LANES,), jnp.float32),
            pltpu.VMEM((NUM_SUBCORES, LANES), jnp.float32),
            pltpu.VMEM((NUM_SUBCORES, LANES), jnp.float32),
            pltpu.VMEM((N_SEGS,), jnp.int32),
        ],
    )(values, prefix_sum)
    return out


def kernel(values, prefix_sum):
    out = _jagged_argmax_sc(values, prefix_sum.astype(jnp.int32))
    return out.astype(jnp.int64)


# speedup vs baseline: 5.9798x; 1.0635x over previous
import jax
import jax.numpy as jnp
from jax import lax
from jax.experimental import pallas as pl
from jax.experimental.pallas import tpu as pltpu
from jax.experimental.pallas import tpu_sc as plsc

N_TOKENS = 32768
N_SEGS = 16
NUM_SUBCORES = 16
CHUNK = N_TOKENS // NUM_SUBCORES
LANES = 16
VREGS = CHUNK // LANES
I32_MAX = jnp.iinfo(jnp.int32).max


def _sc_body(values_hbm, ps_hbm, pv_hbm, pp_hbm, out_hbm,
             vals_v, ps_v, res_v, res_p, buf_v, buf_p, out_v):
    sid = lax.axis_index("s")
    cid = lax.axis_index("c")
    base = sid * CHUNK

    pltpu.sync_copy(values_hbm.at[pl.ds(base, CHUNK)], vals_v)
    pltpu.sync_copy(ps_hbm, ps_v)

    iota = lax.iota(jnp.int32, LANES)
    neg_inf = jnp.float32(-jnp.inf)
    pos_inf = jnp.float32(jnp.inf)
    neg16 = jnp.full((LANES,), neg_inf, jnp.float32)
    inf16 = jnp.full((LANES,), pos_inf, jnp.float32)

    ps16 = ps_v[...]
    seg_max = neg16
    seg_pos = inf16
    for s in range(N_SEGS):
        end_s = ps16[s]
        start_s = ps16[s - 1] if s > 0 else jnp.int32(0)
        lo = jnp.maximum(start_s, base)
        hi = jnp.minimum(end_s, base + CHUNK)
        lo_c = jnp.clip(lo - base, 0, CHUNK)
        hi_c = jnp.clip(hi - base, 0, CHUNK)
        j0 = lo_c // LANES
        j1 = (hi_c + (LANES - 1)) // LANES

        def body(j, carry, lo=lo, hi=hi, start_s=start_s):
            mv, mp = carry
            v = vals_v[pl.ds(j * LANES, LANES)]
            pos = base + j * LANES + iota
            inside = (pos >= lo) & (pos < hi)
            v = jnp.where(inside, v, neg_inf)
            upd = v > mv
            mp = jnp.where(upd, (pos - start_s).astype(jnp.float32), mp)
            mv = jnp.where(upd, v, mv)
            return mv, mp

        mv, mp = lax.fori_loop(j0, j1, body, (neg16, inf16))
        m = mv
        for k in (8, 4, 2, 1):
            m = jnp.maximum(m, m.at[iota ^ k].get(mode="promise_in_bounds"))
        p = jnp.where(mv == m, mp, inf16)
        for k in (8, 4, 2, 1):
            p = jnp.minimum(p, p.at[iota ^ k].get(mode="promise_in_bounds"))
        seg_max = jnp.where(iota == s, m, seg_max)
        seg_pos = jnp.where(iota == s, p, seg_pos)

    res_v[...] = seg_max
    res_p[...] = seg_pos
    pltpu.sync_copy(res_v, pv_hbm.at[sid])
    pltpu.sync_copy(res_p, pp_hbm.at[sid])
    plsc.subcore_barrier()

    @pl.when((sid == 0) & (cid == 0))
    def _():
        pltpu.sync_copy(pv_hbm, buf_v)
        pltpu.sync_copy(pp_hbm, buf_p)
        gmax = neg16
        for i in range(NUM_SUBCORES):
            gmax = jnp.maximum(gmax, buf_v[i, :])
        gpos = inf16
        for i in range(NUM_SUBCORES):
            rv = buf_v[i, :]
            rp = buf_p[i, :]
            gpos = jnp.minimum(gpos, jnp.where(rv == gmax, rp, inf16))
        empty = gmax == neg16
        out_v[...] = jnp.where(empty, jnp.full((LANES,), I32_MAX, jnp.int32),
                               gpos.astype(jnp.int32))
        pltpu.sync_copy(out_v, out_hbm)


@jax.jit
def _jagged_argmax_sc(values, prefix_sum):
    mesh = plsc.VectorSubcoreMesh(
        core_axis_name="c", subcore_axis_name="s",
        num_cores=1, num_subcores=NUM_SUBCORES)
    _, _, out = pl.kernel(
        _sc_body,
        out_type=[
            jax.ShapeDtypeStruct((NUM_SUBCORES, LANES), jnp.float32),
            jax.ShapeDtypeStruct((NUM_SUBCORES, LANES), jnp.float32),
            jax.ShapeDtypeStruct((N_SEGS,), jnp.int32),
        ],
        mesh=mesh,
        scratch_types=[
            pltpu.VMEM((CHUNK,), jnp.float32),
            pltpu.VMEM((N_SEGS,), jnp.int32),
            pltpu.VMEM((LANES,), jnp.float32),
            pltpu.VMEM((LANES,), jnp.float32),
            pltpu.VMEM((NUM_SUBCORES, LANES), jnp.float32),
            pltpu.VMEM((NUM_SUBCORES, LANES), jnp.float32),
            pltpu.VMEM((N_SEGS,), jnp.int32),
        ],
    )(values, prefix_sum)
    return out


def kernel(values, prefix_sum):
    out = _jagged_argmax_sc(values, prefix_sum.astype(jnp.int32))
    return out.astype(jnp.int64)
